# Initial kernel scaffold; baseline (speedup 1.0000x reference)
import jax
import jax.numpy as jnp
from jax.experimental import pallas as pl


def kernel(query_enc_train, query_enc_dev, query_enc_test, ex_ids, split):
    raise NotImplementedError("write your pallas kernel here")


# speedup vs baseline: 1.1417x; 1.1417x over previous
import functools

import jax
import jax.numpy as jnp
from jax import lax
from jax.experimental import pallas as pl
from jax.experimental.pallas import tpu as pltpu
from jax.experimental.pallas import tpu_sc as plsc

VOCAB = 100000
DIM = 128
BATCH = 16384

_info = plsc.get_sparse_core_info()
_NC, _NS, _L = _info.num_cores, _info.num_subcores, _info.num_lanes
_NW = _NC * _NS
_CHUNK = BATCH // _NW
_SUB = 128
_NSUB = _CHUNK // _SUB


def _body(t0, t1, t2, ids_hbm, split_hbm, out_hbm,
          idx_v, spl_v, g0, g1, g2, out_v, s0, s1, s2):
    wid = lax.axis_index("s") * _NC + lax.axis_index("c")
    base = wid * _CHUNK
    pltpu.sync_copy(ids_hbm.at[pl.ds(base, _CHUNK)], idx_v)
    pltpu.sync_copy(split_hbm.at[pl.ds(base, _CHUNK)], spl_v)

    for sb in range(_NSUB):
        lo = sb * _SUB
        idx_slice = idx_v.at[pl.ds(lo, _SUB)]
        c0 = pltpu.async_copy(t0.at[idx_slice], g0, s0)
        c1 = pltpu.async_copy(t1.at[idx_slice], g1, s1)
        c2 = pltpu.async_copy(t2.at[idx_slice], g2, s2)
        c0.wait()
        c1.wait()
        c2.wait()

        def rowgroup(g, _):
            s16 = spl_v[pl.ds(lo + g * _L, _L)]
            for j in range(_L):
                r = g * _L + j
                sj = s16[j]
                w0 = jnp.full((_L,), jnp.where(sj == 0, 1.0, 0.0), jnp.float32)
                w1 = jnp.full((_L,), jnp.where(sj == 1, 1.0, 0.0), jnp.float32)
                w2 = jnp.full((_L,), jnp.where(sj == 2, 1.0, 0.0), jnp.float32)
                for c in range(DIM // _L):
                    sl = pl.ds(c * _L, _L)
                    v0 = g0[r, sl]
                    v1 = g1[r, sl]
                    v2 = g2[r, sl]
                    out_v[r, sl] = v0 * w0 + v1 * w1 + v2 * w2
            return _

        lax.fori_loop(0, _SUB // _L, rowgroup, None)
        pltpu.sync_copy(out_v, out_hbm.at[pl.ds(base + lo, _SUB)])


@jax.jit
def _run(t0, t1, t2, ids, split):
    mesh = plsc.VectorSubcoreMesh(core_axis_name="c", subcore_axis_name="s")
    return pl.kernel(
        _body,
        mesh=mesh,
        out_type=jax.ShapeDtypeStruct((BATCH, DIM), jnp.float32),
        scratch_types=[
            pltpu.VMEM((_CHUNK,), jnp.int32),
            pltpu.VMEM((_CHUNK,), jnp.int32),
            pltpu.VMEM((_SUB, DIM), jnp.float32),
            pltpu.VMEM((_SUB, DIM), jnp.float32),
            pltpu.VMEM((_SUB, DIM), jnp.float32),
            pltpu.VMEM((_SUB, DIM), jnp.float32),
            pltpu.SemaphoreType.DMA,
            pltpu.SemaphoreType.DMA,
            pltpu.SemaphoreType.DMA,
        ],
    )(t0, t1, t2, ids, split)


def kernel(query_enc_train, query_enc_dev, query_enc_test, ex_ids, split):
    return _run(query_enc_train, query_enc_dev, query_enc_test,
                ex_ids.astype(jnp.int32), split.astype(jnp.int32))


# speedup vs baseline: 2.2769x; 1.9944x over previous
import functools

import jax
import jax.numpy as jnp
from jax import lax
from jax.experimental import pallas as pl
from jax.experimental.pallas import tpu as pltpu
from jax.experimental.pallas import tpu_sc as plsc

VOCAB = 100000
DIM = 128
BATCH = 16384

_info = plsc.get_sparse_core_info()
_NC, _NS, _L = _info.num_cores, _info.num_subcores, _info.num_lanes
_NW = _NC * _NS
_CHUNK = BATCH // _NW
_NV = _CHUNK // _L
_PAD = _CHUNK + 3 * _L
_NQ = _PAD // _L


def _body(t0, t1, t2, ids_hbm, split_hbm, out_hbm,
          idx_v, spl_v, sidx_v, spos_v, rows_v, gsem, ssem):
    wid = lax.axis_index("s") * _NC + lax.axis_index("c")
    base = wid * _CHUNK
    pltpu.sync_copy(ids_hbm.at[pl.ds(base, _CHUNK)], idx_v)
    pltpu.sync_copy(split_hbm.at[pl.ds(base, _CHUNK)], spl_v)

    iota = lax.iota(jnp.int32, _L)

    def count(i, c):
        s = spl_v[pl.ds(i * _L, _L)]
        c0, c1 = c
        c0 = c0 + jnp.sum((s == 0).astype(jnp.int32))
        c1 = c1 + jnp.sum((s == 1).astype(jnp.int32))
        return (c0, c1)

    c0, c1 = lax.fori_loop(0, _NV, count, (jnp.int32(0), jnp.int32(0)))
    c2 = _CHUNK - c0 - c1
    b1 = (c0 + _L - 1) & ~(_L - 1)
    b2 = b1 + ((c1 + _L - 1) & ~(_L - 1))
    b3 = b2 + ((c2 + _L - 1) & ~(_L - 1))

    def place(i, o):
        o0, o1, o2 = o
        s = spl_v[pl.ds(i * _L, _L)]
        ids = idx_v[pl.ds(i * _L, _L)]
        pos = base + i * _L + iota
        m0 = (s == 0).astype(jnp.int32)
        m1 = (s == 1).astype(jnp.int32)
        m2 = (s == 2).astype(jnp.int32)
        r0 = plsc.cumsum(m0)
        r1 = plsc.cumsum(m1)
        r2 = plsc.cumsum(m2)
        slot = (m0 * (o0 + r0 - 1) + m1 * (o1 + r1 - 1) + m2 * (o2 + r2 - 1))
        plsc.store_scatter(sidx_v, [slot], ids)
        plsc.store_scatter(spos_v, [slot >> 4, slot & (_L - 1)], pos)
        return (o0 + r0[_L - 1], o1 + r1[_L - 1], o2 + r2[_L - 1])

    o0, o1, o2 = lax.fori_loop(0, _NV, place, (jnp.int32(0), b1, b2))

    for ot, bt, bn in ((o0, jnp.int32(0), b1), (o1, b1, b2), (o2, b2, b3)):
        pcnt = bn - ot
        first_idx = sidx_v[pl.ds(bt, _L)][0]
        first_pos = spos_v[bt >> 4, :][0]
        slots = ot + iota
        mask = iota < pcnt
        plsc.store_scatter(sidx_v, [slots], jnp.full((_L,), first_idx,
                                                     jnp.int32), mask=mask)
        plsc.store_scatter(spos_v, [slots >> 4, slots & (_L - 1)],
                           jnp.full((_L,), first_pos, jnp.int32), mask=mask)

    for tab, qs, qe in ((t0, jnp.int32(0), b1 // _L), (t1, b1 // _L, b2 // _L),
                        (t2, b2 // _L, b3 // _L)):
        def fire(q, _, tab=tab):
            pltpu.async_copy(tab.at[sidx_v.at[pl.ds(q * _L, _L)]],
                             rows_v.at[pl.ds(q * _L, _L)], gsem)
            return _
        lax.fori_loop(qs, qe, fire, 0)

    def drain_g(q, _):
        pltpu.make_async_copy(t0.at[sidx_v.at[pl.ds(0, _L)]],
                              rows_v.at[pl.ds(0, _L)], gsem).wait()
        return _
    lax.fori_loop(0, b3 // _L, drain_g, 0)

    def fire_s(q, _):
        pltpu.async_copy(rows_v.at[pl.ds(q * _L, _L)],
                         out_hbm.at[spos_v.at[q]], ssem)
        return _
    lax.fori_loop(0, b3 // _L, fire_s, 0)

    def drain_s(q, _):
        pltpu.make_async_copy(rows_v.at[pl.ds(0, _L)],
                              out_hbm.at[spos_v.at[0]], ssem).wait()
        return _
    lax.fori_loop(0, b3 // _L, drain_s, 0)


@jax.jit
def _run(t0, t1, t2, ids, split):
    mesh = plsc.VectorSubcoreMesh(core_axis_name="c", subcore_axis_name="s")
    return pl.kernel(
        _body,
        mesh=mesh,
        compiler_params=pltpu.CompilerParams(needs_layout_passes=False),
        out_type=jax.ShapeDtypeStruct((BATCH, DIM), jnp.float32),
        scratch_types=[
            pltpu.VMEM((_CHUNK,), jnp.int32),
            pltpu.VMEM((_CHUNK,), jnp.int32),
            pltpu.VMEM((_PAD,), jnp.int32),
            pltpu.VMEM((_NQ, _L), jnp.int32),
            pltpu.VMEM((_PAD, DIM), jnp.float32),
            pltpu.SemaphoreType.DMA,
            pltpu.SemaphoreType.DMA,
        ],
    )(t0, t1, t2, ids, split)


def kernel(query_enc_train, query_enc_dev, query_enc_test, ex_ids, split):
    return _run(query_enc_train, query_enc_dev, query_enc_test,
                ex_ids.astype(jnp.int32), split.astype(jnp.int32))
